# baseline (speedup 1.0000x reference)
import functools

import jax
import jax.numpy as jnp
from jax import lax
from jax.experimental import pallas as pl
from jax.experimental.pallas import tpu as pltpu
from jax.experimental.pallas import tpu_sc as plsc

N = 10000
E = 320000
D = 128

NPAD = 10240
NAGG = 10112
ROWS_PER_TILE = NAGG // 16
NWORKERS = 32
EPW = E // NWORKERS
C = 40
NCHUNKS = EPW // C
SUPER = 10
NSUPER = NCHUNKS // SUPER
BLK = 256


def _proj_body(x_ref, wk, bk, wq, bq, wv, bv, ws, bb, kd_ref, qv_ref, skip_ref):
    x = x_ref[...]
    kd_ref[...] = jnp.dot(x, wk[...], preferred_element_type=jnp.float32) + bk[...]
    qv_ref[:, : D] = jnp.dot(x, wq[...], preferred_element_type=jnp.float32) + bq[...]
    qv_ref[:, D:] = jnp.dot(x, wv[...], preferred_element_type=jnp.float32) + bv[...]
    skip_ref[...] = jnp.dot(x, ws[...], preferred_element_type=jnp.float32) + bb[...]


def _proj(x_pad, Wk, bk, Wq, bq, Wv, bv, Ws, bb):
    grid = (NPAD // BLK,)
    w_spec = pl.BlockSpec((D, D), lambda i: (0, 0))
    b_spec = pl.BlockSpec((1, D), lambda i: (0, 0))
    return pl.pallas_call(
        _proj_body,
        grid=grid,
        in_specs=[
            pl.BlockSpec((BLK, D), lambda i: (i, 0)),
            w_spec, b_spec, w_spec, b_spec, w_spec, b_spec, w_spec, b_spec,
        ],
        out_specs=[
            pl.BlockSpec((BLK, D), lambda i: (i, 0)),
            pl.BlockSpec((BLK, 2 * D), lambda i: (i, 0)),
            pl.BlockSpec((BLK, D), lambda i: (i, 0)),
        ],
        out_shape=[
            jax.ShapeDtypeStruct((NPAD, D), jnp.float32),
            jax.ShapeDtypeStruct((NPAD, 2 * D), jnp.float32),
            jax.ShapeDtypeStruct((NPAD, D), jnp.float32),
        ],
    )(x_pad, Wk, bk, Wq, bq, Wv, bv, Ws, bb)


def _sc_body(kd, qv, skip, src4, dst4, out, dsti, srci,
             krs, qvs, agg, isem, gsems, ssems):
    cid = lax.axis_index("c")
    sid = lax.axis_index("s")
    wid = sid * 2 + cid
    rbase = sid * ROWS_PER_TILE

    @pl.when(cid == 0)
    def _():
        pltpu.sync_copy(skip.at[pl.ds(rbase, ROWS_PER_TILE)],
                        agg.at[pl.ds(rbase, ROWS_PER_TILE)])

    @pl.when(cid != 0)
    def _():
        zero = jnp.zeros((16,), jnp.float32)

        def zrow(e, carry):
            for j in range(D // 16):
                krs[0][e, pl.ds(j * 16, 16)] = zero
            return carry

        lax.fori_loop(0, C, zrow, 0)
        for r in range(ROWS_PER_TILE // C):
            pltpu.sync_copy(krs[0], agg.at[pl.ds(rbase + r * C, C)])
        rem = ROWS_PER_TILE % C
        if rem:
            pltpu.sync_copy(
                krs[0].at[pl.ds(0, rem)],
                agg.at[pl.ds(rbase + (ROWS_PER_TILE // C) * C, rem)])

    def issue_load_super(k):
        pltpu.async_copy(dst4.at[wid, k], dsti.at[k % 2], isem)
        pltpu.async_copy(src4.at[wid, k], srci.at[k % 2], isem)

    def wait_load_super():
        pltpu.make_async_copy(dst4.at[wid, 0], dsti.at[0], isem).wait()
        pltpu.make_async_copy(src4.at[wid, 0], srci.at[0], isem).wait()

    def issue_gather(c, sg, bk, bq):
        par, row = (c // SUPER) % 2, c % SUPER
        sem = gsems[sg]
        pltpu.async_copy(kd.at[dsti.at[par, row]], krs[bk], sem)
        pltpu.async_copy(qv.at[srci.at[par, row]], qvs[bq], sem)

    def wait_gather(sg, bk, bq):
        sem = gsems[sg]
        pltpu.make_async_copy(kd.at[dsti.at[0, 0]], krs[bk], sem).wait()
        pltpu.make_async_copy(qv.at[srci.at[0, 0]], qvs[bq], sem).wait()

    def issue_scatter(c, bk):
        par, row = (c // SUPER) % 2, c % SUPER
        pltpu.async_copy(krs[bk], agg.at[dsti.at[par, row]], ssems[bk],
                         add=True)

    def wait_scatter(bk):
        pltpu.make_async_copy(krs[bk], agg.at[dsti.at[0, 0]], ssems[bk]).wait()

    HI = jnp.int32(-65536)

    def compute(bk, bq):
        kr, qvr = krs[bk], qvs[bq]
        M = D // 32

        def edge(e, c2):
            qw = [qvr[e, pl.ds(16 * m, 16)] for m in range(M)]
            qlo = [lax.bitcast_convert_type(qw[m] << 16, jnp.float32) for m in range(M)]
            qhi = [lax.bitcast_convert_type(qw[m] & HI, jnp.float32) for m in range(M)]
            kx = [kr[e, pl.ds(16 * g, 16)] for g in range(2 * M)]
            ex = [jnp.exp(-(kx[2 * m + s] + (qlo, qhi)[s][m]))
                  for m in range(M) for s in (0, 1)]
            vw = [qvr[e, pl.ds(64 + 16 * m, 16)] for m in range(M)]
            vlo = [lax.bitcast_convert_type(vw[m] << 16, jnp.float32) for m in range(M)]
            vhi = [lax.bitcast_convert_type(vw[m] & HI, jnp.float32) for m in range(M)]
            eta = [1.0 / (1.0 + ex[g]) for g in range(2 * M)]
            for m in range(M):
                kr[e, pl.ds(32 * m, 16)] = eta[2 * m] * vlo[m]
                kr[e, pl.ds(32 * m + 16, 16)] = eta[2 * m + 1] * vhi[m]
            return c2

        lax.fori_loop(0, C, edge, 0, unroll=2)

    plsc.subcore_barrier()

    issue_load_super(0)
    wait_load_super()
    issue_load_super(1)
    issue_gather(0, 0, 0, 0)
    issue_gather(1, 1, 1, 1)

    def step(c, sk, sq, sg):
        nk, nq = (sk + 2) % 4, (sq + 2) % 3

        @pl.when(jnp.logical_and((c + 2) % SUPER == 0, c + 2 < NCHUNKS))
        def _():
            wait_load_super()

        wait_gather(sg, sk, sq)

        @pl.when(c >= 2)
        def _():
            wait_scatter(nk)

        @pl.when(jnp.logical_and(
            jnp.logical_and(c % SUPER == 1, c >= SUPER),
            c // SUPER + 1 < NSUPER))
        def _():
            issue_load_super(c // SUPER + 1)

        @pl.when(c + 2 < NCHUNKS)
        def _():
            issue_gather(c + 2, sg, nk, nq)

        compute(sk, sq)
        issue_scatter(c, sk)

    def body(i, carry):
        for t in range(12):
            step(12 * i + t, t % 4, t % 3, t % 2)
        return carry

    NTAIL = NCHUNKS % 12
    lax.fori_loop(0, NCHUNKS // 12, body, 0)
    for t in range(NTAIL):
        c = NCHUNKS - NTAIL + t
        step(c, c % 4, c % 3, c % 2)
    wait_scatter((NCHUNKS - 2) % 4)
    wait_scatter((NCHUNKS - 1) % 4)

    plsc.subcore_barrier()
    pltpu.sync_copy(agg.at[pl.ds(rbase, ROWS_PER_TILE)],
                    out.at[cid, pl.ds(rbase, ROWS_PER_TILE)])


@functools.partial(
    pl.kernel,
    mesh=plsc.VectorSubcoreMesh(core_axis_name="c", subcore_axis_name="s"),
    out_type=jax.ShapeDtypeStruct((2, NAGG, D), jnp.float32),
    scratch_types=[
        pltpu.VMEM((2, SUPER, C), jnp.int32),
        pltpu.VMEM((2, SUPER, C), jnp.int32),
        pltpu.VMEM((C, D), jnp.float32),
        pltpu.VMEM((C, D), jnp.float32),
        pltpu.VMEM((C, D), jnp.float32),
        pltpu.VMEM((C, D), jnp.float32),
        pltpu.VMEM((C, D), jnp.int32),
        pltpu.VMEM((C, D), jnp.int32),
        pltpu.VMEM((C, D), jnp.int32),
        pltpu.VMEM_SHARED((NAGG, D), jnp.float32),
        pltpu.SemaphoreType.DMA,
        pltpu.SemaphoreType.DMA,
        pltpu.SemaphoreType.DMA,
        pltpu.SemaphoreType.DMA,
        pltpu.SemaphoreType.DMA,
        pltpu.SemaphoreType.DMA,
        pltpu.SemaphoreType.DMA,
    ],
)
def _sc_msg(kd, qv, skip, src4, dst4, out, dsti, srci,
            kr0, kr1, kr2, kr3, qv0, qv1, qv2, agg,
            isem, g0, g1, s0, s1, s2, s3):
    _sc_body(kd, qv, skip, src4, dst4, out, dsti, srci,
             (kr0, kr1, kr2, kr3), (qv0, qv1, qv2), agg,
             isem, (g0, g1), (s0, s1, s2, s3))


def _add_body(a_ref, b_ref, o_ref):
    o_ref[...] = a_ref[...] + b_ref[...]


def _combine(p0, p1):
    blk = 128
    grid = (NAGG // blk,)
    spec = pl.BlockSpec((blk, D), lambda i: (i, 0))
    return pl.pallas_call(
        _add_body,
        grid=grid,
        in_specs=[spec, spec],
        out_specs=spec,
        out_shape=jax.ShapeDtypeStruct((NAGG, D), jnp.float32),
    )(p0, p1)


def _perm(a):
    n = a.shape[0]
    return a.reshape(n, D // 32, 16, 2).swapaxes(2, 3).reshape(n, D)


def _unperm(a):
    n = a.shape[0]
    return a.reshape(n, D // 32, 2, 16).swapaxes(2, 3).reshape(n, D)


def kernel(x, edge_index, W_key, b_key, W_query, b_query, W_value, b_value,
           W_skip, bias):
    x_pad = jnp.pad(x, ((0, NPAD - N), (0, 0)))
    kd, qv, skip = _proj(
        x_pad,
        W_key, b_key.reshape(1, D),
        W_query, b_query.reshape(1, D),
        W_value, b_value.reshape(1, D),
        W_skip, bias.reshape(1, D),
    )
    qvu = jax.lax.bitcast_convert_type(
        qv.astype(jnp.bfloat16).reshape(NPAD, 2 * D // 2, 2), jnp.int32)
    src4 = edge_index[0].reshape(NWORKERS, NSUPER, SUPER, C)
    dst4 = edge_index[1].reshape(NWORKERS, NSUPER, SUPER, C)
    partials = _sc_msg(_perm(kd), qvu, _perm(skip), src4, dst4)
    out = _combine(partials[0], partials[1])
    return _unperm(out)[:N]


# speedup vs baseline: 1.2657x; 1.2657x over previous
import functools

import jax
import jax.numpy as jnp
from jax import lax
from jax.experimental import pallas as pl
from jax.experimental.pallas import tpu as pltpu
from jax.experimental.pallas import tpu_sc as plsc

N = 10000
E = 320000
D = 128

NPAD = 10240
NAGG = 10112
ROWS_PER_TILE = NAGG // 16
NWORKERS = 32
EPW = E // NWORKERS
C = 40
NCHUNKS = EPW // C
SUPER = 10
NSUPER = NCHUNKS // SUPER
BLK = 256


def _rne16(t):
    u = lax.bitcast_convert_type(t, jnp.uint32)
    return (u + jnp.uint32(0x7FFF) + ((u >> 16) & jnp.uint32(1))) >> 16


def _proj_body(x_ref, wk, bk, wq, bq, wv, bv, ws, bb, kd_ref, qv_ref, skip_ref):
    x = x_ref[...]
    kd_ref[...] = jnp.dot(x, wk[...], preferred_element_type=jnp.float32) + bk[...]
    skip_ref[...] = jnp.dot(x, ws[...], preferred_element_type=jnp.float32) + bb[...]
    q = jnp.dot(x, wq[...], preferred_element_type=jnp.float32) + bq[...]
    v = jnp.dot(x, wv[...], preferred_element_type=jnp.float32) + bv[...]
    qw = _rne16(q[:, : D // 2]) | (_rne16(q[:, D // 2:]) << 16)
    vw = _rne16(v[:, : D // 2]) | (_rne16(v[:, D // 2:]) << 16)
    qv_ref[:, : D // 2] = lax.bitcast_convert_type(qw, jnp.int32)
    qv_ref[:, D // 2:] = lax.bitcast_convert_type(vw, jnp.int32)


def _proj(x_pad, Wk, bk, Wq, bq, Wv, bv, Ws, bb):
    grid = (NPAD // BLK,)
    w_spec = pl.BlockSpec((D, D), lambda i: (0, 0))
    b_spec = pl.BlockSpec((1, D), lambda i: (0, 0))
    return pl.pallas_call(
        _proj_body,
        grid=grid,
        in_specs=[
            pl.BlockSpec((BLK, D), lambda i: (i, 0)),
            w_spec, b_spec, w_spec, b_spec, w_spec, b_spec, w_spec, b_spec,
        ],
        out_specs=[
            pl.BlockSpec((BLK, D), lambda i: (i, 0)),
            pl.BlockSpec((BLK, D), lambda i: (i, 0)),
            pl.BlockSpec((BLK, D), lambda i: (i, 0)),
        ],
        out_shape=[
            jax.ShapeDtypeStruct((NPAD, D), jnp.float32),
            jax.ShapeDtypeStruct((NPAD, D), jnp.int32),
            jax.ShapeDtypeStruct((NPAD, D), jnp.float32),
        ],
    )(x_pad, Wk, bk, Wq, bq, Wv, bv, Ws, bb)


def _sc_body(kd, qv, skip, src4, dst4, out, dsti, srci,
             krs, qvs, agg, isem, gsems, ssems):
    cid = lax.axis_index("c")
    sid = lax.axis_index("s")
    wid = sid * 2 + cid
    rbase = sid * ROWS_PER_TILE

    @pl.when(cid == 0)
    def _():
        pltpu.sync_copy(skip.at[pl.ds(rbase, ROWS_PER_TILE)],
                        agg.at[pl.ds(rbase, ROWS_PER_TILE)])

    @pl.when(cid != 0)
    def _():
        zero = jnp.zeros((16,), jnp.float32)

        def zrow(e, carry):
            for j in range(D // 16):
                krs[0][e, pl.ds(j * 16, 16)] = zero
            return carry

        lax.fori_loop(0, C, zrow, 0)
        for r in range(ROWS_PER_TILE // C):
            pltpu.sync_copy(krs[0], agg.at[pl.ds(rbase + r * C, C)])
        rem = ROWS_PER_TILE % C
        if rem:
            pltpu.sync_copy(
                krs[0].at[pl.ds(0, rem)],
                agg.at[pl.ds(rbase + (ROWS_PER_TILE // C) * C, rem)])

    def issue_load_super(k):
        pltpu.async_copy(dst4.at[wid, k], dsti.at[k % 2], isem)
        pltpu.async_copy(src4.at[wid, k], srci.at[k % 2], isem)

    def wait_load_super():
        pltpu.make_async_copy(dst4.at[wid, 0], dsti.at[0], isem).wait()
        pltpu.make_async_copy(src4.at[wid, 0], srci.at[0], isem).wait()

    def issue_gather(c, sg, bk, bq):
        par, row = (c // SUPER) % 2, c % SUPER
        sem = gsems[sg]
        pltpu.async_copy(kd.at[dsti.at[par, row]], krs[bk], sem)
        pltpu.async_copy(qv.at[srci.at[par, row]], qvs[bq], sem)

    def wait_gather(sg, bk, bq):
        sem = gsems[sg]
        pltpu.make_async_copy(kd.at[dsti.at[0, 0]], krs[bk], sem).wait()
        pltpu.make_async_copy(qv.at[srci.at[0, 0]], qvs[bq], sem).wait()

    def issue_scatter(c, bk):
        par, row = (c // SUPER) % 2, c % SUPER
        pltpu.async_copy(krs[bk], agg.at[dsti.at[par, row]], ssems[bk],
                         add=True)

    def wait_scatter(bk):
        pltpu.make_async_copy(krs[bk], agg.at[dsti.at[0, 0]], ssems[bk]).wait()

    HI = jnp.int32(-65536)

    def compute(bk, bq):
        kr, qvr = krs[bk], qvs[bq]
        M = D // 32
        H = D // 2

        def edge(e, c2):
            qw = [qvr[e, pl.ds(16 * m, 16)] for m in range(M)]
            qlo = [lax.bitcast_convert_type(qw[m] << 16, jnp.float32)
                   for m in range(M)]
            qhi = [lax.bitcast_convert_type(qw[m] & HI, jnp.float32)
                   for m in range(M)]
            klo = [kr[e, pl.ds(16 * m, 16)] for m in range(M)]
            khi = [kr[e, pl.ds(H + 16 * m, 16)] for m in range(M)]
            elo = [jnp.exp(-(klo[m] + qlo[m])) for m in range(M)]
            ehi = [jnp.exp(-(khi[m] + qhi[m])) for m in range(M)]
            vw = [qvr[e, pl.ds(H + 16 * m, 16)] for m in range(M)]
            vlo = [lax.bitcast_convert_type(vw[m] << 16, jnp.float32)
                   for m in range(M)]
            vhi = [lax.bitcast_convert_type(vw[m] & HI, jnp.float32)
                   for m in range(M)]
            tlo = [1.0 / (1.0 + elo[m]) for m in range(M)]
            thi = [1.0 / (1.0 + ehi[m]) for m in range(M)]
            for m in range(M):
                kr[e, pl.ds(16 * m, 16)] = tlo[m] * vlo[m]
                kr[e, pl.ds(H + 16 * m, 16)] = thi[m] * vhi[m]
            return c2

        lax.fori_loop(0, C, edge, 0, unroll=2)

    plsc.subcore_barrier()

    issue_load_super(0)
    wait_load_super()
    issue_load_super(1)
    issue_gather(0, 0, 0, 0)
    issue_gather(1, 1, 1, 1)

    def step(c, sk, sq, sg):
        nk, nq = (sk + 2) % 4, (sq + 2) % 3

        @pl.when(jnp.logical_and((c + 2) % SUPER == 0, c + 2 < NCHUNKS))
        def _():
            wait_load_super()

        wait_gather(sg, sk, sq)

        @pl.when(c >= 2)
        def _():
            wait_scatter(nk)

        @pl.when(jnp.logical_and(
            jnp.logical_and(c % SUPER == 1, c >= SUPER),
            c // SUPER + 1 < NSUPER))
        def _():
            issue_load_super(c // SUPER + 1)

        @pl.when(c + 2 < NCHUNKS)
        def _():
            issue_gather(c + 2, sg, nk, nq)

        compute(sk, sq)
        issue_scatter(c, sk)

    def body(i, carry):
        for t in range(12):
            step(12 * i + t, t % 4, t % 3, t % 2)
        return carry

    NTAIL = NCHUNKS % 12
    lax.fori_loop(0, NCHUNKS // 12, body, 0)
    for t in range(NTAIL):
        c = NCHUNKS - NTAIL + t
        step(c, c % 4, c % 3, c % 2)
    wait_scatter((NCHUNKS - 2) % 4)
    wait_scatter((NCHUNKS - 1) % 4)

    plsc.subcore_barrier()
    pltpu.sync_copy(agg.at[pl.ds(rbase, ROWS_PER_TILE)],
                    out.at[cid, pl.ds(rbase, ROWS_PER_TILE)])


@functools.partial(
    pl.kernel,
    mesh=plsc.VectorSubcoreMesh(core_axis_name="c", subcore_axis_name="s"),
    out_type=jax.ShapeDtypeStruct((2, NAGG, D), jnp.float32),
    scratch_types=[
        pltpu.VMEM((2, SUPER, C), jnp.int32),
        pltpu.VMEM((2, SUPER, C), jnp.int32),
        pltpu.VMEM((C, D), jnp.float32),
        pltpu.VMEM((C, D), jnp.float32),
        pltpu.VMEM((C, D), jnp.float32),
        pltpu.VMEM((C, D), jnp.float32),
        pltpu.VMEM((C, D), jnp.int32),
        pltpu.VMEM((C, D), jnp.int32),
        pltpu.VMEM((C, D), jnp.int32),
        pltpu.VMEM_SHARED((NAGG, D), jnp.float32),
        pltpu.SemaphoreType.DMA,
        pltpu.SemaphoreType.DMA,
        pltpu.SemaphoreType.DMA,
        pltpu.SemaphoreType.DMA,
        pltpu.SemaphoreType.DMA,
        pltpu.SemaphoreType.DMA,
        pltpu.SemaphoreType.DMA,
    ],
)
def _sc_msg(kd, qv, skip, src4, dst4, out, dsti, srci,
            kr0, kr1, kr2, kr3, qv0, qv1, qv2, agg,
            isem, g0, g1, s0, s1, s2, s3):
    _sc_body(kd, qv, skip, src4, dst4, out, dsti, srci,
             (kr0, kr1, kr2, kr3), (qv0, qv1, qv2), agg,
             isem, (g0, g1), (s0, s1, s2, s3))


def _add_body(a_ref, b_ref, o_ref):
    o_ref[...] = a_ref[...] + b_ref[...]


def _combine(p0, p1):
    blk = 128
    grid = (NAGG // blk,)
    spec = pl.BlockSpec((blk, D), lambda i: (i, 0))
    return pl.pallas_call(
        _add_body,
        grid=grid,
        in_specs=[spec, spec],
        out_specs=spec,
        out_shape=jax.ShapeDtypeStruct((NAGG, D), jnp.float32),
    )(p0, p1)


def kernel(x, edge_index, W_key, b_key, W_query, b_query, W_value, b_value,
           W_skip, bias):
    x_pad = jnp.pad(x, ((0, NPAD - N), (0, 0)))
    kd, qvu, skip = _proj(
        x_pad,
        W_key, b_key.reshape(1, D),
        W_query, b_query.reshape(1, D),
        W_value, b_value.reshape(1, D),
        W_skip, bias.reshape(1, D),
    )
    src4 = edge_index[0].reshape(NWORKERS, NSUPER, SUPER, C)
    dst4 = edge_index[1].reshape(NWORKERS, NSUPER, SUPER, C)
    partials = _sc_msg(kd, qvu, skip, src4, dst4)
    out = _combine(partials[0], partials[1])
    return out[:N]


# speedup vs baseline: 1.3041x; 1.0303x over previous
import functools

import jax
import jax.numpy as jnp
from jax import lax
from jax.experimental import pallas as pl
from jax.experimental.pallas import tpu as pltpu
from jax.experimental.pallas import tpu_sc as plsc

N = 10000
E = 320000
D = 128

NAGG = 10112
ROWS_PER_TILE = NAGG // 16
NWORKERS = 32
EPW = E // NWORKERS
C = 40
NCHUNKS = EPW // C
SUPER = 10
NSUPER = NCHUNKS // SUPER
BLK = 128


def _rne16(t):
    u = lax.bitcast_convert_type(t, jnp.uint32)
    return (u + jnp.uint32(0x7FFF) + ((u >> 16) & jnp.uint32(1))) >> 16


def _proj_body(x_ref, wk, bk, wq, bq, wv, bv, ws, bb, kd_ref, qv_ref, skip_ref):
    x = x_ref[...]
    kd_ref[...] = jnp.dot(x, wk[...], preferred_element_type=jnp.float32) + bk[...]
    skip_ref[...] = jnp.dot(x, ws[...], preferred_element_type=jnp.float32) + bb[...]
    q = jnp.dot(x, wq[...], preferred_element_type=jnp.float32) + bq[...]
    v = jnp.dot(x, wv[...], preferred_element_type=jnp.float32) + bv[...]
    qw = _rne16(q[:, : D // 2]) | (_rne16(q[:, D // 2:]) << 16)
    vw = _rne16(v[:, : D // 2]) | (_rne16(v[:, D // 2:]) << 16)
    qv_ref[:, : D // 2] = lax.bitcast_convert_type(qw, jnp.int32)
    qv_ref[:, D // 2:] = lax.bitcast_convert_type(vw, jnp.int32)


def _proj(x_in, Wk, bk, Wq, bq, Wv, bv, Ws, bb):
    grid = (NAGG // BLK,)
    w_spec = pl.BlockSpec((D, D), lambda i: (0, 0))
    b_spec = pl.BlockSpec((1, D), lambda i: (0, 0))
    return pl.pallas_call(
        _proj_body,
        grid=grid,
        in_specs=[
            pl.BlockSpec((BLK, D), lambda i: (i, 0)),
            w_spec, b_spec, w_spec, b_spec, w_spec, b_spec, w_spec, b_spec,
        ],
        out_specs=[
            pl.BlockSpec((BLK, D), lambda i: (i, 0)),
            pl.BlockSpec((BLK, D), lambda i: (i, 0)),
            pl.BlockSpec((BLK, D), lambda i: (i, 0)),
        ],
        out_shape=[
            jax.ShapeDtypeStruct((NAGG, D), jnp.float32),
            jax.ShapeDtypeStruct((NAGG, D), jnp.int32),
            jax.ShapeDtypeStruct((NAGG, D), jnp.float32),
        ],
    )(x_in, Wk, bk, Wq, bq, Wv, bv, Ws, bb)


def _sc_body(kd, qv, skip, src4, dst4, out, dsti, srci,
             krs, qvs, agg, isem, gsems, ssems):
    cid = lax.axis_index("c")
    sid = lax.axis_index("s")
    wid = sid * 2 + cid
    rbase = sid * ROWS_PER_TILE

    @pl.when(cid == 0)
    def _():
        pltpu.sync_copy(skip.at[pl.ds(rbase, ROWS_PER_TILE)],
                        agg.at[pl.ds(rbase, ROWS_PER_TILE)])

    @pl.when(cid != 0)
    def _():
        zero = jnp.zeros((16,), jnp.float32)

        def zrow(e, carry):
            for j in range(D // 16):
                krs[0][e, pl.ds(j * 16, 16)] = zero
            return carry

        lax.fori_loop(0, C, zrow, 0)
        for r in range(ROWS_PER_TILE // C):
            pltpu.sync_copy(krs[0], agg.at[pl.ds(rbase + r * C, C)])
        rem = ROWS_PER_TILE % C
        if rem:
            pltpu.sync_copy(
                krs[0].at[pl.ds(0, rem)],
                agg.at[pl.ds(rbase + (ROWS_PER_TILE // C) * C, rem)])

    def issue_load_super(k):
        pltpu.async_copy(dst4.at[wid, k], dsti.at[k % 2], isem)
        pltpu.async_copy(src4.at[wid, k], srci.at[k % 2], isem)

    def wait_load_super():
        pltpu.make_async_copy(dst4.at[wid, 0], dsti.at[0], isem).wait()
        pltpu.make_async_copy(src4.at[wid, 0], srci.at[0], isem).wait()

    def issue_gather(c, sg, bk, bq):
        par, row = (c // SUPER) % 2, c % SUPER
        sem = gsems[sg]
        pltpu.async_copy(kd.at[dsti.at[par, row]], krs[bk], sem)
        pltpu.async_copy(qv.at[srci.at[par, row]], qvs[bq], sem)

    def wait_gather(sg, bk, bq):
        sem = gsems[sg]
        pltpu.make_async_copy(kd.at[dsti.at[0, 0]], krs[bk], sem).wait()
        pltpu.make_async_copy(qv.at[srci.at[0, 0]], qvs[bq], sem).wait()

    def issue_scatter(c, bk):
        par, row = (c // SUPER) % 2, c % SUPER
        pltpu.async_copy(krs[bk], agg.at[dsti.at[par, row]], ssems[bk],
                         add=True)

    def wait_scatter(bk):
        pltpu.make_async_copy(krs[bk], agg.at[dsti.at[0, 0]], ssems[bk]).wait()

    HI = jnp.int32(-65536)

    def compute(bk, bq):
        kr, qvr = krs[bk], qvs[bq]
        M = D // 32
        H = D // 2

        def edge(e, c2):
            qw = [qvr[e, pl.ds(16 * m, 16)] for m in range(M)]
            qlo = [lax.bitcast_convert_type(qw[m] << 16, jnp.float32)
                   for m in range(M)]
            qhi = [lax.bitcast_convert_type(qw[m] & HI, jnp.float32)
                   for m in range(M)]
            klo = [kr[e, pl.ds(16 * m, 16)] for m in range(M)]
            khi = [kr[e, pl.ds(H + 16 * m, 16)] for m in range(M)]
            elo = [jnp.exp(-(klo[m] + qlo[m])) for m in range(M)]
            ehi = [jnp.exp(-(khi[m] + qhi[m])) for m in range(M)]
            vw = [qvr[e, pl.ds(H + 16 * m, 16)] for m in range(M)]
            vlo = [lax.bitcast_convert_type(vw[m] << 16, jnp.float32)
                   for m in range(M)]
            vhi = [lax.bitcast_convert_type(vw[m] & HI, jnp.float32)
                   for m in range(M)]
            tlo = [1.0 / (1.0 + elo[m]) for m in range(M)]
            thi = [1.0 / (1.0 + ehi[m]) for m in range(M)]
            for m in range(M):
                kr[e, pl.ds(16 * m, 16)] = tlo[m] * vlo[m]
                kr[e, pl.ds(H + 16 * m, 16)] = thi[m] * vhi[m]
            return c2

        lax.fori_loop(0, C, edge, 0, unroll=2)

    plsc.subcore_barrier()

    issue_load_super(0)
    wait_load_super()
    issue_load_super(1)
    issue_gather(0, 0, 0, 0)
    issue_gather(1, 1, 1, 1)

    def step(c, sk, sq, sg):
        nk, nq = (sk + 2) % 4, (sq + 2) % 3

        @pl.when(jnp.logical_and((c + 2) % SUPER == 0, c + 2 < NCHUNKS))
        def _():
            wait_load_super()

        wait_gather(sg, sk, sq)

        @pl.when(c >= 2)
        def _():
            wait_scatter(nk)

        @pl.when(jnp.logical_and(
            jnp.logical_and(c % SUPER == 1, c >= SUPER),
            c // SUPER + 1 < NSUPER))
        def _():
            issue_load_super(c // SUPER + 1)

        @pl.when(c + 2 < NCHUNKS)
        def _():
            issue_gather(c + 2, sg, nk, nq)

        compute(sk, sq)
        issue_scatter(c, sk)

    def body(i, carry):
        for t in range(12):
            step(12 * i + t, t % 4, t % 3, t % 2)
        return carry

    NTAIL = NCHUNKS % 12
    lax.fori_loop(0, NCHUNKS // 12, body, 0)
    for t in range(NTAIL):
        c = NCHUNKS - NTAIL + t
        step(c, c % 4, c % 3, c % 2)
    wait_scatter((NCHUNKS - 2) % 4)
    wait_scatter((NCHUNKS - 1) % 4)

    plsc.subcore_barrier()
    pltpu.sync_copy(agg.at[pl.ds(rbase, ROWS_PER_TILE)],
                    out.at[cid, pl.ds(rbase, ROWS_PER_TILE)])


@functools.partial(
    pl.kernel,
    mesh=plsc.VectorSubcoreMesh(core_axis_name="c", subcore_axis_name="s"),
    out_type=jax.ShapeDtypeStruct((2, NAGG, D), jnp.float32),
    scratch_types=[
        pltpu.VMEM((2, SUPER, C), jnp.int32),
        pltpu.VMEM((2, SUPER, C), jnp.int32),
        pltpu.VMEM((C, D), jnp.float32),
        pltpu.VMEM((C, D), jnp.float32),
        pltpu.VMEM((C, D), jnp.float32),
        pltpu.VMEM((C, D), jnp.float32),
        pltpu.VMEM((C, D), jnp.int32),
        pltpu.VMEM((C, D), jnp.int32),
        pltpu.VMEM((C, D), jnp.int32),
        pltpu.VMEM_SHARED((NAGG, D), jnp.float32),
        pltpu.SemaphoreType.DMA,
        pltpu.SemaphoreType.DMA,
        pltpu.SemaphoreType.DMA,
        pltpu.SemaphoreType.DMA,
        pltpu.SemaphoreType.DMA,
        pltpu.SemaphoreType.DMA,
        pltpu.SemaphoreType.DMA,
    ],
)
def _sc_msg(kd, qv, skip, src4, dst4, out, dsti, srci,
            kr0, kr1, kr2, kr3, qv0, qv1, qv2, agg,
            isem, g0, g1, s0, s1, s2, s3):
    _sc_body(kd, qv, skip, src4, dst4, out, dsti, srci,
             (kr0, kr1, kr2, kr3), (qv0, qv1, qv2), agg,
             isem, (g0, g1), (s0, s1, s2, s3))


def _add_body(a_ref, b_ref, o_ref):
    o_ref[...] = a_ref[...] + b_ref[...]


def _combine(p0, p1):
    blk = 400
    grid = (N // blk,)
    spec = pl.BlockSpec((blk, D), lambda i: (i, 0))
    return pl.pallas_call(
        _add_body,
        grid=grid,
        in_specs=[spec, spec],
        out_specs=spec,
        out_shape=jax.ShapeDtypeStruct((N, D), jnp.float32),
    )(p0, p1)


def kernel(x, edge_index, W_key, b_key, W_query, b_query, W_value, b_value,
           W_skip, bias):
    kd, qvu, skip = _proj(
        x,
        W_key, b_key.reshape(1, D),
        W_query, b_query.reshape(1, D),
        W_value, b_value.reshape(1, D),
        W_skip, bias.reshape(1, D),
    )
    src4 = edge_index[0].reshape(NWORKERS, NSUPER, SUPER, C)
    dst4 = edge_index[1].reshape(NWORKERS, NSUPER, SUPER, C)
    partials = _sc_msg(kd, qvu, skip, src4, dst4)
    return _combine(partials[0], partials[1])
